# baseline (speedup 1.0000x reference)
import dataclasses
import functools

import jax
import jax.numpy as jnp
from jax import lax
from jax.experimental import pallas as pl
from jax.experimental.pallas import tpu as pltpu
from jax.experimental.pallas import tpu_sc as plsc

N = 10000
E = 320000
H = 128
NP = 10240
NW = 32
EP = 327680
EPW = EP // NW
NCH = EPW // 128
RPT = NP // 16

_MESH = dict(core_axis_name="c", subcore_axis_name="s")

_SC_PARAMS = pltpu.CompilerParams()
if "needs_layout_passes" in pltpu.CompilerParams.__dataclass_fields__:
    _SC_PARAMS = dataclasses.replace(_SC_PARAMS, needs_layout_passes=False)


def _sc_degree(dst2d):
    mesh = plsc.VectorSubcoreMesh(**_MESH)

    @functools.partial(
        pl.kernel,
        out_type=jax.ShapeDtypeStruct((NW, NP), jnp.float32),
        mesh=mesh,
        compiler_params=_SC_PARAMS,
        scratch_types=[
            pltpu.VMEM((NCH, 128), jnp.int32),
            pltpu.VMEM((NP,), jnp.float32),
        ],
    )
    def k(dst_hbm, out_hbm, idx_v, deg_v):
        c = lax.axis_index("c")
        s = lax.axis_index("s")
        wid = c * 16 + s
        pltpu.sync_copy(dst_hbm.at[pl.ds(wid * NCH, NCH)], idx_v)
        zeros = jnp.zeros((16,), jnp.float32)

        @pl.loop(0, NP, step=16)
        def _(i):
            deg_v[pl.ds(i, 16)] = zeros

        ones = jnp.ones((16,), jnp.float32)

        @pl.loop(0, NCH)
        def _(kk):
            @pl.loop(0, 128, step=16)
            def _(j):
                idx = idx_v[kk, pl.ds(j, 16)]
                plsc.addupdate_scatter(deg_v, [idx], ones)

        pltpu.sync_copy(deg_v, out_hbm.at[wid])

    return k(dst2d)


NCHT = (EP // 128) // 16
HIDX = NCHT // 4


def _sc_scatter(g, src2d, dst2d):
    mesh = plsc.VectorSubcoreMesh(**_MESH)

    @functools.partial(
        pl.kernel,
        out_type=jax.ShapeDtypeStruct((NP, H), jnp.float32),
        mesh=mesh,
        compiler_params=_SC_PARAMS,
        scratch_types=[
            pltpu.VMEM((HIDX, 128), jnp.int32),
            pltpu.VMEM((HIDX, 128), jnp.int32),
            pltpu.VMEM((128, H), jnp.float32),
            pltpu.VMEM((128, H), jnp.float32),
            pltpu.VMEM_SHARED((NP, H), jnp.float32),
            pltpu.SemaphoreType.DMA,
            pltpu.SemaphoreType.DMA,
            pltpu.SemaphoreType.DMA,
            pltpu.SemaphoreType.DMA,
            pltpu.SemaphoreType.DMA,
        ],
    )
    def k(g_hbm, src_hbm, dst_hbm, out_hbm, sidx, didx, rowsA, rowsB, acc,
          gsA, gsB, ssA, ssB, ws):
        c = lax.axis_index("c")
        s = lax.axis_index("s")
        zeros = jnp.zeros((16,), jnp.float32)

        @pl.when(c == 0)
        def _():
            @pl.loop(0, 128)
            def _(r):
                @pl.loop(0, H, step=16)
                def _(j):
                    rowsA[r, pl.ds(j, 16)] = zeros

            for j in range(RPT // 128):
                pltpu.async_copy(rowsA, acc.at[pl.ds(s * RPT + j * 128, 128)],
                                 ws)
            for j in range(RPT // 128):
                pltpu.make_async_copy(rowsA, acc.at[pl.ds(s * RPT, 128)],
                                      ws).wait()

        plsc.subcore_barrier()

        def gather(kk, buf, sem):
            pltpu.async_copy(g_hbm.at[sidx.at[kk]], buf, sem)

        def wait_gather(buf, sem):
            pltpu.make_async_copy(g_hbm.at[sidx.at[0]], buf, sem).wait()

        def scat(kk, buf, sem):
            pltpu.async_copy(buf, acc.at[didx.at[kk]], sem, add=True)

        def wait_scat(buf, sem):
            pltpu.make_async_copy(buf, acc.at[didx.at[0]], sem).wait()

        pairs = HIDX // 2

        for h in range(4):
            base = pl.multiple_of(s * NCHT + h * HIDX, 8)

            @pl.when(c == 0)
            def _():
                pltpu.sync_copy(src_hbm.at[pl.ds(base, HIDX)], sidx)
                pltpu.sync_copy(dst_hbm.at[pl.ds(base, HIDX)], didx)
                gather(0, rowsA, gsA)

                @pl.loop(0, pairs)
                def _(t):
                    k0 = 2 * t
                    wait_gather(rowsA, gsA)
                    scat(k0, rowsA, ssA)

                    @pl.when(t > 0)
                    def _():
                        wait_scat(rowsB, ssB)

                    gather(k0 + 1, rowsB, gsB)
                    wait_gather(rowsB, gsB)
                    scat(k0 + 1, rowsB, ssB)
                    wait_scat(rowsA, ssA)

                    @pl.when(t < pairs - 1)
                    def _():
                        gather(k0 + 2, rowsA, gsA)

                wait_scat(rowsB, ssB)

        plsc.subcore_barrier()

        @pl.when(c == 0)
        def _():
            for j in range(RPT // 128):
                buf, sem = (rowsA, gsA) if j % 2 == 0 else (rowsB, gsB)
                if j >= 2:
                    pltpu.make_async_copy(
                        buf, out_hbm.at[pl.ds(s * RPT, 128)], sem).wait()
                pltpu.sync_copy(acc.at[pl.ds(s * RPT + j * 128, 128)], buf)
                pltpu.async_copy(buf, out_hbm.at[pl.ds(s * RPT + j * 128, 128)],
                                 sem)
            pltpu.make_async_copy(rowsA, out_hbm.at[pl.ds(s * RPT, 128)],
                                  gsA).wait()
            pltpu.make_async_copy(rowsB, out_hbm.at[pl.ds(s * RPT, 128)],
                                  gsB).wait()

    return k(g, src2d, dst2d)


_R = 512


def _tc_prep(xp, embp, degp, W0):
    def body(x_ref, emb_ref, deg_ref, w_ref, hw_ref, g_ref, dinv_ref):
        xa = x_ref[...]
        onehot = (xa == lax.broadcasted_iota(jnp.int32, (_R, 128), 1)
                  ).astype(jnp.float32)
        deg = jnp.sum(deg_ref[...], axis=0) + 1.0
        dinv = lax.rsqrt(deg).reshape(_R, 1)
        h0 = jnp.dot(onehot, emb_ref[...], preferred_element_type=jnp.float32)
        hw = jnp.dot(h0, w_ref[...], preferred_element_type=jnp.float32)
        hw_ref[...] = hw
        g_ref[...] = hw * dinv
        dinv_ref[...] = dinv

    return pl.pallas_call(
        body,
        grid=(NP // _R,),
        in_specs=[
            pl.BlockSpec((_R, 1), lambda i: (i, 0)),
            pl.BlockSpec((128, 128), lambda i: (0, 0)),
            pl.BlockSpec((NW, _R), lambda i: (0, i)),
            pl.BlockSpec((128, 128), lambda i: (0, 0)),
        ],
        out_specs=[
            pl.BlockSpec((_R, H), lambda i: (i, 0)),
            pl.BlockSpec((_R, H), lambda i: (i, 0)),
            pl.BlockSpec((_R, 1), lambda i: (i, 0)),
        ],
        out_shape=[
            jax.ShapeDtypeStruct((NP, H), jnp.float32),
            jax.ShapeDtypeStruct((NP, H), jnp.float32),
            jax.ShapeDtypeStruct((NP, 1), jnp.float32),
        ],
    )(xp, embp, degp, W0)


def _tc_mid(s_part, hwp, dinv, b, W):
    def body(s0_ref, hwp_ref, dinv_ref, b_ref, w_ref, hw_ref, g_ref):
        dinv = dinv_ref[...]
        agg = s0_ref[...] * dinv
        agg = agg + hwp_ref[...] * (dinv * dinv) + b_ref[...]
        h = jnp.maximum(agg, 0.0)
        hw = jnp.dot(h, w_ref[...], preferred_element_type=jnp.float32)
        hw_ref[...] = hw
        g_ref[...] = hw * dinv

    return pl.pallas_call(
        body,
        grid=(NP // _R,),
        in_specs=[
            pl.BlockSpec((_R, H), lambda i: (i, 0)),
            pl.BlockSpec((_R, H), lambda i: (i, 0)),
            pl.BlockSpec((_R, 1), lambda i: (i, 0)),
            pl.BlockSpec((1, H), lambda i: (0, 0)),
            pl.BlockSpec((H, H), lambda i: (0, 0)),
        ],
        out_specs=[
            pl.BlockSpec((_R, H), lambda i: (i, 0)),
            pl.BlockSpec((_R, H), lambda i: (i, 0)),
        ],
        out_shape=[
            jax.ShapeDtypeStruct((NP, H), jnp.float32),
            jax.ShapeDtypeStruct((NP, H), jnp.float32),
        ],
    )(s_part, hwp, dinv, b, W)


def _tc_readout(s_part, hwp, dinv, b, RW0, Rb0, RW1, Rb1, RW2, Rb2):
    def body(s0_ref, hwp_ref, dinv_ref, b_ref,
             w0_ref, b0_ref, w1_ref, b1_ref, w2_ref, b2_ref, out_ref):
        dinv = dinv_ref[...]
        agg = s0_ref[...] * dinv
        h = agg + hwp_ref[...] * (dinv * dinv) + b_ref[...]
        y = jnp.maximum(
            jnp.dot(h, w0_ref[...], preferred_element_type=jnp.float32)
            + b0_ref[...], 0.0)
        y = jnp.maximum(
            jnp.dot(y, w1_ref[...], preferred_element_type=jnp.float32)
            + b1_ref[...], 0.0)
        out_ref[...] = (
            jnp.dot(y, w2_ref[...], preferred_element_type=jnp.float32)
            + b2_ref[...])

    return pl.pallas_call(
        body,
        grid=(NP // _R,),
        in_specs=[
            pl.BlockSpec((_R, H), lambda i: (i, 0)),
            pl.BlockSpec((_R, H), lambda i: (i, 0)),
            pl.BlockSpec((_R, 1), lambda i: (i, 0)),
            pl.BlockSpec((1, H), lambda i: (0, 0)),
            pl.BlockSpec((H, 64), lambda i: (0, 0)),
            pl.BlockSpec((1, 64), lambda i: (0, 0)),
            pl.BlockSpec((64, 32), lambda i: (0, 0)),
            pl.BlockSpec((1, 32), lambda i: (0, 0)),
            pl.BlockSpec((32, H), lambda i: (0, 0)),
            pl.BlockSpec((1, H), lambda i: (0, 0)),
        ],
        out_specs=pl.BlockSpec((_R, H), lambda i: (i, 0)),
        out_shape=jax.ShapeDtypeStruct((NP, H), jnp.float32),
    )(s_part, hwp, dinv, b,
      RW0, Rb0, RW1, Rb1, RW2, Rb2)


def kernel(x, edge_index, emb, W0, b0, W1, b1, W2, b2,
           RW0, Rb0, RW1, Rb1, RW2, Rb2):
    x = x.astype(jnp.int32)
    ei = edge_index.astype(jnp.int32)
    ep_load = EP
    pad = ep_load - E
    src = jnp.concatenate([ei[0], jnp.zeros((pad,), jnp.int32)])
    dst = jnp.concatenate([ei[1], jnp.full((pad,), N, jnp.int32)])
    src2d = src.reshape(ep_load // 128, 128)
    dst2d = dst.reshape(ep_load // 128, 128)
    xp = jnp.concatenate([x, jnp.zeros((NP - N,), jnp.int32)]).reshape(NP, 1)
    embp = jnp.pad(emb, ((0, 128 - emb.shape[0]), (0, 0)))

    degp = _sc_degree(dst2d)
    hw, g, dinv = _tc_prep(xp, embp, degp, W0)
    for b_prev, W in ((b0, W1), (b1, W2)):
        s_part = _sc_scatter(g, src2d, dst2d)
        hw, g = _tc_mid(s_part, hw, dinv, b_prev.reshape(1, H), W)
    s_part = _sc_scatter(g, src2d, dst2d)
    out = _tc_readout(s_part, hw, dinv, b2.reshape(1, H),
                      RW0, Rb0.reshape(1, 64), RW1, Rb1.reshape(1, 32),
                      RW2, Rb2.reshape(1, H))
    return out[:N]


# speedup vs baseline: 1.9791x; 1.9791x over previous
import dataclasses
import functools

import jax
import jax.numpy as jnp
from jax import lax
from jax.experimental import pallas as pl
from jax.experimental.pallas import tpu as pltpu
from jax.experimental.pallas import tpu_sc as plsc

N = 10000
E = 320000
H = 128
NP = 10240
NW = 32
EP = 327680
EPW = EP // NW
NCH = EPW // 128
RPT = NP // 16

_MESH = dict(core_axis_name="c", subcore_axis_name="s")

_SC_PARAMS = pltpu.CompilerParams()
if "needs_layout_passes" in pltpu.CompilerParams.__dataclass_fields__:
    _SC_PARAMS = dataclasses.replace(_SC_PARAMS, needs_layout_passes=False)


def _sc_degree(dst2d):
    mesh = plsc.VectorSubcoreMesh(**_MESH)

    @functools.partial(
        pl.kernel,
        out_type=jax.ShapeDtypeStruct((NW, NP), jnp.float32),
        mesh=mesh,
        compiler_params=_SC_PARAMS,
        scratch_types=[
            pltpu.VMEM((NCH, 128), jnp.int32),
            pltpu.VMEM((NP,), jnp.float32),
        ],
    )
    def k(dst_hbm, out_hbm, idx_v, deg_v):
        c = lax.axis_index("c")
        s = lax.axis_index("s")
        wid = c * 16 + s
        pltpu.sync_copy(dst_hbm.at[pl.ds(wid * NCH, NCH)], idx_v)
        zeros = jnp.zeros((16,), jnp.float32)

        @pl.loop(0, NP, step=16)
        def _(i):
            deg_v[pl.ds(i, 16)] = zeros

        ones = jnp.ones((16,), jnp.float32)

        @pl.loop(0, NCH)
        def _(kk):
            @pl.loop(0, 128, step=16)
            def _(j):
                idx = idx_v[kk, pl.ds(j, 16)]
                plsc.addupdate_scatter(deg_v, [idx], ones)

        pltpu.sync_copy(deg_v, out_hbm.at[wid])

    return k(dst2d)


NCHT = (EP // 128) // 16
HIDX = NCHT // 4


def _sc_scatter(g, src2d, dst2d):
    mesh = plsc.VectorSubcoreMesh(**_MESH)

    @functools.partial(
        pl.kernel,
        out_type=jax.ShapeDtypeStruct((NP, H), jnp.float32),
        mesh=mesh,
        compiler_params=_SC_PARAMS,
        scratch_types=[
            pltpu.VMEM((HIDX, 128), jnp.int32),
            pltpu.VMEM((HIDX, 128), jnp.int32),
            pltpu.VMEM((128, H), jnp.float32),
            pltpu.VMEM((128, H), jnp.float32),
            pltpu.VMEM_SHARED((NP, H), jnp.float32),
            pltpu.SemaphoreType.DMA,
            pltpu.SemaphoreType.DMA,
            pltpu.SemaphoreType.DMA,
            pltpu.SemaphoreType.DMA,
            pltpu.SemaphoreType.DMA,
        ],
    )
    def k(g_hbm, src_hbm, dst_hbm, out_hbm, sidx, didx, rowsA, rowsB, acc,
          gsA, gsB, ssA, ssB, ws):
        c = lax.axis_index("c")
        s = lax.axis_index("s")
        zeros = jnp.zeros((16,), jnp.float32)

        @pl.when(c == 0)
        def _():
            @pl.loop(0, 128)
            def _(r):
                @pl.loop(0, H, step=16)
                def _(j):
                    rowsA[r, pl.ds(j, 16)] = zeros

            for j in range(RPT // 128):
                pltpu.async_copy(rowsA, acc.at[pl.ds(s * RPT + j * 128, 128)],
                                 ws)
            for j in range(RPT // 128):
                pltpu.make_async_copy(rowsA, acc.at[pl.ds(s * RPT, 128)],
                                      ws).wait()

        plsc.subcore_barrier()

        def gather(kk, buf, sem):
            pltpu.async_copy(g_hbm.at[sidx.at[kk]], buf, sem)

        def wait_gather(buf, sem):
            pltpu.make_async_copy(g_hbm.at[sidx.at[0]], buf, sem).wait()

        def scat(kk, buf, sem):
            pltpu.async_copy(buf, acc.at[didx.at[kk]], sem, add=True)

        def wait_scat(buf, sem):
            pltpu.make_async_copy(buf, acc.at[didx.at[0]], sem).wait()

        pairs = HIDX // 2

        for h in range(4):
            base = pl.multiple_of(s * NCHT + h * HIDX, 8)

            @pl.when(c == 0)
            def _():
                pltpu.sync_copy(src_hbm.at[pl.ds(base, HIDX)], sidx)
                pltpu.sync_copy(dst_hbm.at[pl.ds(base, HIDX)], didx)
                gather(0, rowsA, gsA)

                @pl.loop(0, pairs)
                def _(t):
                    k0 = 2 * t
                    wait_gather(rowsA, gsA)
                    scat(k0, rowsA, ssA)

                    @pl.when(t > 0)
                    def _():
                        wait_scat(rowsB, ssB)

                    gather(k0 + 1, rowsB, gsB)
                    wait_gather(rowsB, gsB)
                    scat(k0 + 1, rowsB, ssB)
                    wait_scat(rowsA, ssA)

                    @pl.when(t < pairs - 1)
                    def _():
                        gather(k0 + 2, rowsA, gsA)

                wait_scat(rowsB, ssB)

        plsc.subcore_barrier()

        @pl.when(c == 0)
        def _():
            for j in range(RPT // 128):
                buf, sem = (rowsA, gsA) if j % 2 == 0 else (rowsB, gsB)
                if j >= 2:
                    pltpu.make_async_copy(
                        buf, out_hbm.at[pl.ds(s * RPT, 128)], sem).wait()
                pltpu.sync_copy(acc.at[pl.ds(s * RPT + j * 128, 128)], buf)
                pltpu.async_copy(buf, out_hbm.at[pl.ds(s * RPT + j * 128, 128)],
                                 sem)
            pltpu.make_async_copy(rowsA, out_hbm.at[pl.ds(s * RPT, 128)],
                                  gsA).wait()
            pltpu.make_async_copy(rowsB, out_hbm.at[pl.ds(s * RPT, 128)],
                                  gsB).wait()

    return k(g, src2d, dst2d)


_R = 512


def _tc_prep(xp, embp, degp, W0):
    def body(x_ref, emb_ref, deg_ref, w_ref, hw_ref, g_ref, dinv_ref):
        xa = x_ref[...]
        onehot = (xa == lax.broadcasted_iota(jnp.int32, (_R, 128), 1)
                  ).astype(jnp.float32)
        deg = jnp.sum(deg_ref[...], axis=0) + 1.0
        dinv = lax.rsqrt(deg).reshape(_R, 1)
        h0 = jnp.dot(onehot, emb_ref[...], preferred_element_type=jnp.float32)
        hw = jnp.dot(h0, w_ref[...], preferred_element_type=jnp.float32)
        hw_ref[...] = hw
        g_ref[...] = hw * dinv
        dinv_ref[...] = dinv

    return pl.pallas_call(
        body,
        grid=(NP // _R,),
        in_specs=[
            pl.BlockSpec((_R, 1), lambda i: (i, 0)),
            pl.BlockSpec((128, 128), lambda i: (0, 0)),
            pl.BlockSpec((NW, _R), lambda i: (0, i)),
            pl.BlockSpec((128, 128), lambda i: (0, 0)),
        ],
        out_specs=[
            pl.BlockSpec((_R, H), lambda i: (i, 0)),
            pl.BlockSpec((_R, H), lambda i: (i, 0)),
            pl.BlockSpec((_R, 1), lambda i: (i, 0)),
        ],
        out_shape=[
            jax.ShapeDtypeStruct((NP, H), jnp.float32),
            jax.ShapeDtypeStruct((NP, H), jnp.float32),
            jax.ShapeDtypeStruct((NP, 1), jnp.float32),
        ],
    )(xp, embp, degp, W0)


def _tc_mid(s_part, hwp, dinv, b, W):
    def body(s0_ref, hwp_ref, dinv_ref, b_ref, w_ref, hw_ref, g_ref):
        dinv = dinv_ref[...]
        agg = s0_ref[...] * dinv
        agg = agg + hwp_ref[...] * (dinv * dinv) + b_ref[...]
        h = jnp.maximum(agg, 0.0)
        hw = jnp.dot(h, w_ref[...], preferred_element_type=jnp.float32)
        hw_ref[...] = hw
        g_ref[...] = hw * dinv

    return pl.pallas_call(
        body,
        grid=(NP // _R,),
        in_specs=[
            pl.BlockSpec((_R, H), lambda i: (i, 0)),
            pl.BlockSpec((_R, H), lambda i: (i, 0)),
            pl.BlockSpec((_R, 1), lambda i: (i, 0)),
            pl.BlockSpec((1, H), lambda i: (0, 0)),
            pl.BlockSpec((H, H), lambda i: (0, 0)),
        ],
        out_specs=[
            pl.BlockSpec((_R, H), lambda i: (i, 0)),
            pl.BlockSpec((_R, H), lambda i: (i, 0)),
        ],
        out_shape=[
            jax.ShapeDtypeStruct((NP, H), jnp.float32),
            jax.ShapeDtypeStruct((NP, H), jnp.float32),
        ],
    )(s_part, hwp, dinv, b, W)


def _tc_readout(s_part, hwp, dinv, b, RW0, Rb0, RW1, Rb1, RW2, Rb2):
    def body(s0_ref, hwp_ref, dinv_ref, b_ref,
             w0_ref, b0_ref, w1_ref, b1_ref, w2_ref, b2_ref, out_ref):
        dinv = dinv_ref[...]
        agg = s0_ref[...] * dinv
        h = agg + hwp_ref[...] * (dinv * dinv) + b_ref[...]
        y = jnp.maximum(
            jnp.dot(h, w0_ref[...], preferred_element_type=jnp.float32)
            + b0_ref[...], 0.0)
        y = jnp.maximum(
            jnp.dot(y, w1_ref[...], preferred_element_type=jnp.float32)
            + b1_ref[...], 0.0)
        out_ref[...] = (
            jnp.dot(y, w2_ref[...], preferred_element_type=jnp.float32)
            + b2_ref[...])

    return pl.pallas_call(
        body,
        grid=(NP // _R,),
        in_specs=[
            pl.BlockSpec((_R, H), lambda i: (i, 0)),
            pl.BlockSpec((_R, H), lambda i: (i, 0)),
            pl.BlockSpec((_R, 1), lambda i: (i, 0)),
            pl.BlockSpec((1, H), lambda i: (0, 0)),
            pl.BlockSpec((H, 64), lambda i: (0, 0)),
            pl.BlockSpec((1, 64), lambda i: (0, 0)),
            pl.BlockSpec((64, 32), lambda i: (0, 0)),
            pl.BlockSpec((1, 32), lambda i: (0, 0)),
            pl.BlockSpec((32, H), lambda i: (0, 0)),
            pl.BlockSpec((1, H), lambda i: (0, 0)),
        ],
        out_specs=pl.BlockSpec((_R, H), lambda i: (i, 0)),
        out_shape=jax.ShapeDtypeStruct((NP, H), jnp.float32),
    )(s_part, hwp, dinv, b,
      RW0, Rb0, RW1, Rb1, RW2, Rb2)


def kernel(x, edge_index, emb, W0, b0, W1, b1, W2, b2,
           RW0, Rb0, RW1, Rb1, RW2, Rb2):
    x = x.astype(jnp.int32)
    ei = edge_index.astype(jnp.int32)
    ep_load = EP
    pad = ep_load - E
    pad_iota = jnp.arange(pad, dtype=jnp.int32)
    src = jnp.concatenate([ei[0], pad_iota % N])
    dst = jnp.concatenate([ei[1], 10016 + pad_iota % (NP - 10016)])
    src2d = src.reshape(ep_load // 128, 128)
    dst2d = dst.reshape(ep_load // 128, 128)
    xp = jnp.concatenate([x, jnp.zeros((NP - N,), jnp.int32)]).reshape(NP, 1)
    embp = jnp.pad(emb, ((0, 128 - emb.shape[0]), (0, 0)))

    degp = _sc_degree(dst2d)
    hw, g, dinv = _tc_prep(xp, embp, degp, W0)
    for b_prev, W in ((b0, W1), (b1, W2)):
        s_part = _sc_scatter(g, src2d, dst2d)
        hw, g = _tc_mid(s_part, hw, dinv, b_prev.reshape(1, H), W)
    s_part = _sc_scatter(g, src2d, dst2d)
    out = _tc_readout(s_part, hw, dinv, b2.reshape(1, H),
                      RW0, Rb0.reshape(1, 64), RW1, Rb1.reshape(1, 32),
                      RW2, Rb2.reshape(1, H))
    return out[:N]


# speedup vs baseline: 3.1126x; 1.5728x over previous
import dataclasses
import functools

import jax
import jax.numpy as jnp
from jax import lax
from jax.experimental import pallas as pl
from jax.experimental.pallas import tpu as pltpu
from jax.experimental.pallas import tpu_sc as plsc

N = 10000
E = 320000
H = 128
NP = 10240
NW = 32
EP = 327680
EPW = EP // NW
NCH = EPW // 128
RPT = NP // 16

_MESH = dict(core_axis_name="c", subcore_axis_name="s")

_SC_PARAMS = pltpu.CompilerParams()
if "needs_layout_passes" in pltpu.CompilerParams.__dataclass_fields__:
    _SC_PARAMS = dataclasses.replace(_SC_PARAMS, needs_layout_passes=False)


def _sc_degree(dst2d):
    mesh = plsc.VectorSubcoreMesh(**_MESH)

    @functools.partial(
        pl.kernel,
        out_type=jax.ShapeDtypeStruct((NW, NP), jnp.float32),
        mesh=mesh,
        compiler_params=_SC_PARAMS,
        scratch_types=[
            pltpu.VMEM((NCH, 128), jnp.int32),
            pltpu.VMEM((NP,), jnp.float32),
        ],
    )
    def k(dst_hbm, out_hbm, idx_v, deg_v):
        c = lax.axis_index("c")
        s = lax.axis_index("s")
        wid = c * 16 + s
        pltpu.sync_copy(dst_hbm.at[pl.ds(wid * NCH, NCH)], idx_v)
        zeros = jnp.zeros((16,), jnp.float32)

        @pl.loop(0, NP, step=16)
        def _(i):
            deg_v[pl.ds(i, 16)] = zeros

        ones = jnp.ones((16,), jnp.float32)

        @pl.loop(0, NCH)
        def _(kk):
            @pl.loop(0, 128, step=16)
            def _(j):
                idx = idx_v[kk, pl.ds(j, 16)]
                plsc.addupdate_scatter(deg_v, [idx], ones)

        pltpu.sync_copy(deg_v, out_hbm.at[wid])

    return k(dst2d)


NCHT = (EP // 128) // 32
HIDX = NCHT // 2


def _sc_scatter(g, src2d, dst2d):
    mesh = plsc.VectorSubcoreMesh(**_MESH)

    @functools.partial(
        pl.kernel,
        out_type=jax.ShapeDtypeStruct((2, NP, H), jnp.float32),
        mesh=mesh,
        compiler_params=_SC_PARAMS,
        scratch_types=[
            pltpu.VMEM((HIDX, 128), jnp.int32),
            pltpu.VMEM((HIDX, 128), jnp.int32),
            pltpu.VMEM((128, H), jnp.float32),
            pltpu.VMEM((128, H), jnp.float32),
            pltpu.VMEM_SHARED((NP, H), jnp.float32),
            pltpu.SemaphoreType.DMA,
            pltpu.SemaphoreType.DMA,
            pltpu.SemaphoreType.DMA,
            pltpu.SemaphoreType.DMA,
            pltpu.SemaphoreType.DMA,
        ],
    )
    def k(g_hbm, src_hbm, dst_hbm, out_hbm, sidx, didx, rowsA, rowsB, acc,
          gsA, gsB, ssA, ssB, ws):
        c = lax.axis_index("c")
        s = lax.axis_index("s")
        wid = c * 16 + s
        zeros = jnp.zeros((16,), jnp.float32)

        @pl.loop(0, 128)
        def _(r):
            @pl.loop(0, H, step=16)
            def _(j):
                rowsA[r, pl.ds(j, 16)] = zeros

        for j in range(RPT // 128):
            pltpu.async_copy(rowsA, acc.at[pl.ds(s * RPT + j * 128, 128)], ws)
        for j in range(RPT // 128):
            pltpu.make_async_copy(rowsA, acc.at[pl.ds(s * RPT, 128)],
                                  ws).wait()

        plsc.subcore_barrier()

        def gather(kk, buf, sem):
            pltpu.async_copy(g_hbm.at[sidx.at[kk]], buf, sem)

        def wait_gather(buf, sem):
            pltpu.make_async_copy(g_hbm.at[sidx.at[0]], buf, sem).wait()

        def scat(kk, buf, sem):
            pltpu.async_copy(buf, acc.at[didx.at[kk]], sem, add=True)

        def wait_scat(buf, sem):
            pltpu.make_async_copy(buf, acc.at[didx.at[0]], sem).wait()

        pairs = HIDX // 2

        for h in range(2):
            base = pl.multiple_of(wid * NCHT + h * HIDX, 8)
            pltpu.sync_copy(src_hbm.at[pl.ds(base, HIDX)], sidx)
            pltpu.sync_copy(dst_hbm.at[pl.ds(base, HIDX)], didx)
            gather(0, rowsA, gsA)

            @pl.loop(0, pairs)
            def _(t):
                k0 = 2 * t
                wait_gather(rowsA, gsA)
                scat(k0, rowsA, ssA)

                @pl.when(t > 0)
                def _():
                    wait_scat(rowsB, ssB)

                gather(k0 + 1, rowsB, gsB)
                wait_gather(rowsB, gsB)
                scat(k0 + 1, rowsB, ssB)
                wait_scat(rowsA, ssA)

                @pl.when(t < pairs - 1)
                def _():
                    gather(k0 + 2, rowsA, gsA)

            wait_scat(rowsB, ssB)

        plsc.subcore_barrier()

        for j in range(RPT // 128):
            buf, sem = (rowsA, gsA) if j % 2 == 0 else (rowsB, gsB)
            if j >= 2:
                pltpu.make_async_copy(
                    buf, out_hbm.at[c, pl.ds(s * RPT, 128)], sem).wait()
            pltpu.sync_copy(acc.at[pl.ds(s * RPT + j * 128, 128)], buf)
            pltpu.async_copy(buf, out_hbm.at[c, pl.ds(s * RPT + j * 128, 128)],
                             sem)
        pltpu.make_async_copy(rowsA, out_hbm.at[c, pl.ds(s * RPT, 128)],
                              gsA).wait()
        pltpu.make_async_copy(rowsB, out_hbm.at[c, pl.ds(s * RPT, 128)],
                              gsB).wait()

    return k(g, src2d, dst2d)


_R = 512


def _tc_prep(xp, embp, degp, W0):
    def body(x_ref, emb_ref, deg_ref, w_ref, hw_ref, g_ref, dinv_ref):
        xa = x_ref[...]
        onehot = (xa == lax.broadcasted_iota(jnp.int32, (_R, 128), 1)
                  ).astype(jnp.float32)
        deg = jnp.sum(deg_ref[...], axis=0) + 1.0
        dinv = lax.rsqrt(deg).reshape(_R, 1)
        h0 = jnp.dot(onehot, emb_ref[...], preferred_element_type=jnp.float32)
        hw = jnp.dot(h0, w_ref[...], preferred_element_type=jnp.float32)
        hw_ref[...] = hw
        g_ref[...] = hw * dinv
        dinv_ref[...] = dinv

    return pl.pallas_call(
        body,
        grid=(NP // _R,),
        in_specs=[
            pl.BlockSpec((_R, 1), lambda i: (i, 0)),
            pl.BlockSpec((128, 128), lambda i: (0, 0)),
            pl.BlockSpec((NW, _R), lambda i: (0, i)),
            pl.BlockSpec((128, 128), lambda i: (0, 0)),
        ],
        out_specs=[
            pl.BlockSpec((_R, H), lambda i: (i, 0)),
            pl.BlockSpec((_R, H), lambda i: (i, 0)),
            pl.BlockSpec((_R, 1), lambda i: (i, 0)),
        ],
        out_shape=[
            jax.ShapeDtypeStruct((NP, H), jnp.float32),
            jax.ShapeDtypeStruct((NP, H), jnp.float32),
            jax.ShapeDtypeStruct((NP, 1), jnp.float32),
        ],
    )(xp, embp, degp, W0)


def _tc_mid(s_part, hwp, dinv, b, W):
    def body(s0_ref, s1_ref, hwp_ref, dinv_ref, b_ref, w_ref, hw_ref, g_ref):
        dinv = dinv_ref[...]
        agg = (s0_ref[...] + s1_ref[...]) * dinv
        agg = agg + hwp_ref[...] * (dinv * dinv) + b_ref[...]
        h = jnp.maximum(agg, 0.0)
        hw = jnp.dot(h, w_ref[...], preferred_element_type=jnp.float32)
        hw_ref[...] = hw
        g_ref[...] = hw * dinv

    return pl.pallas_call(
        body,
        grid=(NP // _R,),
        in_specs=[
            pl.BlockSpec((_R, H), lambda i: (i, 0)),
            pl.BlockSpec((_R, H), lambda i: (i, 0)),
            pl.BlockSpec((_R, H), lambda i: (i, 0)),
            pl.BlockSpec((_R, 1), lambda i: (i, 0)),
            pl.BlockSpec((1, H), lambda i: (0, 0)),
            pl.BlockSpec((H, H), lambda i: (0, 0)),
        ],
        out_specs=[
            pl.BlockSpec((_R, H), lambda i: (i, 0)),
            pl.BlockSpec((_R, H), lambda i: (i, 0)),
        ],
        out_shape=[
            jax.ShapeDtypeStruct((NP, H), jnp.float32),
            jax.ShapeDtypeStruct((NP, H), jnp.float32),
        ],
    )(s_part[0], s_part[1], hwp, dinv, b, W)


def _tc_readout(s_part, hwp, dinv, b, RW0, Rb0, RW1, Rb1, RW2, Rb2):
    def body(s0_ref, s1_ref, hwp_ref, dinv_ref, b_ref,
             w0_ref, b0_ref, w1_ref, b1_ref, w2_ref, b2_ref, out_ref):
        dinv = dinv_ref[...]
        agg = (s0_ref[...] + s1_ref[...]) * dinv
        h = agg + hwp_ref[...] * (dinv * dinv) + b_ref[...]
        y = jnp.maximum(
            jnp.dot(h, w0_ref[...], preferred_element_type=jnp.float32)
            + b0_ref[...], 0.0)
        y = jnp.maximum(
            jnp.dot(y, w1_ref[...], preferred_element_type=jnp.float32)
            + b1_ref[...], 0.0)
        out_ref[...] = (
            jnp.dot(y, w2_ref[...], preferred_element_type=jnp.float32)
            + b2_ref[...])

    return pl.pallas_call(
        body,
        grid=(NP // _R,),
        in_specs=[
            pl.BlockSpec((_R, H), lambda i: (i, 0)),
            pl.BlockSpec((_R, H), lambda i: (i, 0)),
            pl.BlockSpec((_R, H), lambda i: (i, 0)),
            pl.BlockSpec((_R, 1), lambda i: (i, 0)),
            pl.BlockSpec((1, H), lambda i: (0, 0)),
            pl.BlockSpec((H, 64), lambda i: (0, 0)),
            pl.BlockSpec((1, 64), lambda i: (0, 0)),
            pl.BlockSpec((64, 32), lambda i: (0, 0)),
            pl.BlockSpec((1, 32), lambda i: (0, 0)),
            pl.BlockSpec((32, H), lambda i: (0, 0)),
            pl.BlockSpec((1, H), lambda i: (0, 0)),
        ],
        out_specs=pl.BlockSpec((_R, H), lambda i: (i, 0)),
        out_shape=jax.ShapeDtypeStruct((NP, H), jnp.float32),
    )(s_part[0], s_part[1], hwp, dinv, b,
      RW0, Rb0, RW1, Rb1, RW2, Rb2)


def kernel(x, edge_index, emb, W0, b0, W1, b1, W2, b2,
           RW0, Rb0, RW1, Rb1, RW2, Rb2):
    x = x.astype(jnp.int32)
    ei = edge_index.astype(jnp.int32)
    ep_load = EP
    pad = ep_load - E
    pad_iota = jnp.arange(pad, dtype=jnp.int32)
    src = jnp.concatenate([ei[0], pad_iota % N])
    dst = jnp.concatenate([ei[1], 10016 + pad_iota % (NP - 10016)])
    src2d = src.reshape(ep_load // 128, 128)
    dst2d = dst.reshape(ep_load // 128, 128)
    xp = jnp.concatenate([x, jnp.zeros((NP - N,), jnp.int32)]).reshape(NP, 1)
    embp = jnp.pad(emb, ((0, 128 - emb.shape[0]), (0, 0)))

    degp = _sc_degree(dst2d)
    hw, g, dinv = _tc_prep(xp, embp, degp, W0)
    for b_prev, W in ((b0, W1), (b1, W2)):
        s_part = _sc_scatter(g, src2d, dst2d)
        hw, g = _tc_mid(s_part, hw, dinv, b_prev.reshape(1, H), W)
    s_part = _sc_scatter(g, src2d, dst2d)
    out = _tc_readout(s_part, hw, dinv, b2.reshape(1, H),
                      RW0, Rb0.reshape(1, 64), RW1, Rb1.reshape(1, 32),
                      RW2, Rb2.reshape(1, H))
    return out[:N]
